# baseline (speedup 1.0000x reference)
import functools

import numpy as np
import jax
import jax.numpy as jnp
from jax import lax
from jax.experimental import pallas as pl
from jax.experimental.pallas import tpu as pltpu
from jax.experimental.pallas import tpu_sc as plsc

NH = 8
NP = 4
ML = 6
D = 32
C = 256
_SS = np.array([[64, 64], [32, 32], [16, 16], [64, 64], [32, 32], [16, 16]], np.int64)
_STARTS = np.concatenate([np.zeros(1, np.int64), np.cumsum(_SS[:, 0] * _SS[:, 1])[:-1]])
NV = int((_SS[:, 0] * _SS[:, 1]).sum())
PTS = NH * ML * NP
QB = 512
VB = 768

_pt = np.arange(PTS)
_h_of = _pt // (ML * NP)
_l_of = (_pt % (ML * NP)) // NP
_Wc = _SS[_l_of, 1].astype(np.float32)[None]
_Hc = _SS[_l_of, 0].astype(np.float32)[None]
_Wci = _SS[_l_of, 1].astype(np.int32)[None]
_Hci = _SS[_l_of, 0].astype(np.int32)[None]
_invW = (1.0 / _SS[_l_of, 1]).astype(np.float32)[None]
_invH = (1.0 / _SS[_l_of, 0]).astype(np.float32)[None]
_addc = (_STARTS[_l_of] * NH + _h_of).astype(np.int32)[None]
_Sx = np.zeros((2 * ML, PTS), np.float32)
_Sx[2 * _l_of, _pt] = 1.0
_Sy = np.zeros((2 * ML, PTS), np.float32)
_Sy[2 * _l_of + 1, _pt] = 1.0
_Bsum = (_h_of[:, None] == _h_of[None, :]).astype(np.float32)

_CORNERS = ((0, 0), (1, 0), (0, 1), (1, 1))
_FC = np.concatenate([_Wc, _Hc, _invW, _invH], axis=0)
_IC = np.concatenate([_Wci, _Hci, _addc, _Wci * NH], axis=0)
_Ppad = np.zeros((PTS, C), np.float32)
_Ppad[_pt, _h_of * 32 + (_pt % (ML * NP))] = 1.0


def _prep_math(q, ref12, wsox, bsox, wsoy, bsoy, waw, baw, b,
               bsum, sx, sy, fc, ic):
    sox = jnp.dot(q, wsox, preferred_element_type=jnp.float32) + bsox
    soy = jnp.dot(q, wsoy, preferred_element_type=jnp.float32) + bsoy
    logits = jnp.dot(q, waw, preferred_element_type=jnp.float32) + baw
    e = jnp.exp(logits)
    denom = jnp.dot(e, bsum, preferred_element_type=jnp.float32, precision=lax.Precision.HIGHEST)
    aw = e / denom
    refx = jnp.dot(ref12, sx, preferred_element_type=jnp.float32, precision=lax.Precision.HIGHEST)
    refy = jnp.dot(ref12, sy, preferred_element_type=jnp.float32, precision=lax.Precision.HIGHEST)
    Wc, Hc, invW, invH = fc[0:1], fc[1:2], fc[2:3], fc[3:4]
    Wci, Hci, addc0, Wci8 = ic[0:1], ic[1:2], ic[2:3], ic[3:4]
    x = (refx + sox * invW) * Wc - 0.5
    y = (refy + soy * invH) * Hc - 0.5
    x0f = jnp.floor(x)
    fx = x - x0f
    x0 = x0f.astype(jnp.int32)
    y0f = jnp.floor(y)
    fy = y - y0f
    y0 = y0f.astype(jnp.int32)
    vx0 = (x0f >= 0.0) & (x0f <= Wc - 1.0)
    vx1 = (x0f >= -1.0) & (x0f <= Wc - 2.0)
    vy0 = (y0f >= 0.0) & (y0f <= Hc - 1.0)
    vy1 = (y0f >= -1.0) & (y0f <= Hc - 2.0)
    wx0 = jnp.where(vx0, 1.0 - fx, 0.0)
    wx1 = jnp.where(vx1, fx, 0.0)
    awy0 = jnp.where(vy0, 1.0 - fy, 0.0) * aw
    awy1 = jnp.where(vy1, fy, 0.0) * aw
    addc = addc0 + b * np.int32(NV * NH)
    rt0 = jnp.clip(y0, 0, Hci - 1) * Wci8 + addc
    rt1 = jnp.clip(y0 + 1, 0, Hci - 1) * Wci8 + addc
    ct0 = jnp.left_shift(jnp.clip(x0, 0, Wci - 1), 3)
    ct1 = jnp.left_shift(jnp.clip(x0 + 1, 0, Wci - 1), 3)
    idx_c = [rt0 + ct0, rt0 + ct1, rt1 + ct0, rt1 + ct1]
    w_c = [wx0 * awy0, wx1 * awy0, wx0 * awy1, wx1 * awy1]
    return idx_c, w_c


def _prep_kernel(qr, qpr, refr, wsox, bsox, wsoy, bsoy, waw, baw,
                 bsum, sx, sy, fc, ic, ppad, idx_o, w_o0, w_o1, w_o2, w_o3):
    b = pl.program_id(0)
    q = qr[0] + qpr[0]
    idx_c, w_c = _prep_math(
        q, refr[0], wsox[...], bsox[...], wsoy[...], bsoy[...], waw[...],
        baw[...], b, bsum[...], sx[...], sy[...], fc[...], ic[...])
    w_os = (w_o0, w_o1, w_o2, w_o3)
    for c in range(4):
        idx_o[0, :, c, :] = idx_c[c]
        w_os[c][0] = jnp.dot(w_c[c], ppad[...],
                             preferred_element_type=jnp.float32)


def _prep_call(query, query_pos, ref12, wsox, bsox, wsoy, bsoy, waw, baw,
               qoff, nqh):
    bs = query.shape[0]
    ob = qoff // QB
    qspec = pl.BlockSpec((1, QB, C), lambda b, i: (b, i + ob, 0))
    wspec = pl.BlockSpec((C, PTS), lambda b, i: (0, 0))
    bspec = pl.BlockSpec((1, PTS), lambda b, i: (0, 0))
    cspec = pl.BlockSpec((4, PTS), lambda b, i: (0, 0))
    return pl.pallas_call(
        _prep_kernel,
        grid=(bs, nqh // QB),
        in_specs=[
            qspec, qspec,
            pl.BlockSpec((1, QB, 2 * ML), lambda b, i: (b, i + ob, 0)),
            wspec, bspec, wspec, bspec, wspec, bspec,
            pl.BlockSpec((PTS, PTS), lambda b, i: (0, 0)),
            pl.BlockSpec((2 * ML, PTS), lambda b, i: (0, 0)),
            pl.BlockSpec((2 * ML, PTS), lambda b, i: (0, 0)),
            cspec, cspec,
            pl.BlockSpec((PTS, C), lambda b, i: (0, 0)),
        ],
        out_specs=[
            pl.BlockSpec((1, QB, 4, PTS), lambda b, i: (b, i, 0, 0)),
        ] + [pl.BlockSpec((1, QB, C), lambda b, i: (b, i, 0))] * 4,
        out_shape=[
            jax.ShapeDtypeStruct((bs, nqh, 4, PTS), jnp.int32),
        ] + [jax.ShapeDtypeStruct((bs, nqh, C), jnp.float32)] * 4,
    )(query, query_pos, ref12, wsox, bsox, wsoy, bsoy, waw, baw,
      jnp.asarray(_Bsum), jnp.asarray(_Sx), jnp.asarray(_Sy),
      jnp.asarray(_FC), jnp.asarray(_IC), jnp.asarray(_Ppad))


def _vproj_kernel(vr, wv, bv, o):
    o[0] = (jnp.dot(vr[0], wv[...], preferred_element_type=jnp.float32)
            + bv[...]).astype(jnp.bfloat16)


def _vproj_call(value, W_v, b_v):
    bs, nv, _ = value.shape
    return pl.pallas_call(
        _vproj_kernel,
        grid=(bs, nv // VB),
        in_specs=[
            pl.BlockSpec((1, VB, C), lambda b, i: (b, i, 0)),
            pl.BlockSpec((C, C), lambda b, i: (0, 0)),
            pl.BlockSpec((1, C), lambda b, i: (0, 0)),
        ],
        out_specs=pl.BlockSpec((1, VB, C), lambda b, i: (b, i, 0)),
        out_shape=jax.ShapeDtypeStruct((bs, nv, C), jnp.bfloat16),
    )(value, W_v, b_v.reshape(1, C))


def _final_kernel(ar, wo, bo, qr, o):
    o[0] = (jnp.dot(ar[0], wo[...], preferred_element_type=jnp.float32)
            + bo[...] + qr[0])


def _final_call(attn, W_o, b_o, query, qoff, nqh):
    bs = query.shape[0]
    ob = qoff // QB
    spec = pl.BlockSpec((1, QB, C), lambda b, i: (b, i, 0))
    return pl.pallas_call(
        _final_kernel,
        grid=(bs, nqh // QB),
        in_specs=[
            spec,
            pl.BlockSpec((C, C), lambda b, i: (0, 0)),
            pl.BlockSpec((1, C), lambda b, i: (0, 0)),
            pl.BlockSpec((1, QB, C), lambda b, i: (b, i + ob, 0)),
        ],
        out_specs=spec,
        out_shape=jax.ShapeDtypeStruct((bs, nqh, C), jnp.float32),
    )(attn, W_o, b_o.reshape(1, C), query)


def _sc_sample(table, idxr, wrs, bs_nq):
    CQ = 4
    RPT = bs_nq // 32
    NCH = RPT // CQ
    ROWS = CQ * 4 * PTS
    mesh = plsc.VectorSubcoreMesh(core_axis_name="c", subcore_axis_name="s")

    @functools.partial(
        pl.kernel,
        out_type=jax.ShapeDtypeStruct((bs_nq * C,), jnp.float32),
        mesh=mesh,
        compiler_params=pltpu.CompilerParams(use_tc_tiling_on_sc=False,
                                             needs_layout_passes=False),
        scratch_types=[
            pltpu.VMEM((2, ROWS, D), jnp.bfloat16),
            pltpu.VMEM((4, ROWS), jnp.int32),
            pltpu.VMEM((4, 4, CQ * C), jnp.float32),
            pltpu.VMEM((2, CQ * C), jnp.float32),
            pltpu.SemaphoreType.DMA,
            pltpu.SemaphoreType.DMA,
            pltpu.SemaphoreType.DMA,
            pltpu.SemaphoreType.DMA,
            pltpu.SemaphoreType.DMA,
            pltpu.SemaphoreType.DMA,
            pltpu.SemaphoreType.DMA,
            pltpu.SemaphoreType.DMA,
        ],
    )
    def k(table_h, idx_h, w0_h, w1_h, w2_h, w3_h, out_h, gbuf, ibuf, wbuf,
          obuf, si0, si1, si2, si3, sg0, sg1, so0, so1):
        w_hs = (w0_h, w1_h, w2_h, w3_h)
        sems_i = (si0, si1, si2, si3)
        sems_g = (sg0, sg1)
        sems_o = (so0, so1)
        wid = lax.axis_index("s") * 2 + lax.axis_index("c")
        base_row = wid * RPT

        def idx_issue(ch, islot):
            pltpu.async_copy(idx_h.at[pl.ds((base_row + CQ * ch) * 768, ROWS)],
                             ibuf.at[islot], sems_i[islot])
            for cc in range(4):
                pltpu.async_copy(
                    w_hs[cc].at[pl.ds((base_row + CQ * ch) * C, CQ * C)],
                    wbuf.at[islot, cc], sems_i[islot])

        def idx_wait(islot):
            pltpu.make_async_copy(idx_h.at[pl.ds(0, ROWS)],
                                  ibuf.at[islot], sems_i[islot]).wait()
            for cc in range(4):
                pltpu.make_async_copy(w_hs[cc].at[pl.ds(0, 2 * C)],
                                      wbuf.at[islot, cc],
                                      sems_i[islot]).wait()

        def gather_issue(gslot, islot):
            def gi(kk, _):
                pltpu.async_copy(table_h.at[ibuf.at[islot, pl.ds(kk * 128, 128)]],
                                 gbuf.at[gslot, pl.ds(kk * 128, 128), :],
                                 sems_g[gslot])
                return 0
            lax.fori_loop(0, ROWS // 128, gi, 0)

        def gather_wait(gslot):
            pltpu.make_async_copy(table_h.at[pl.ds(0, ROWS), :],
                                  gbuf.at[gslot], sems_g[gslot]).wait()

        def out_issue(ch, oslot):
            pltpu.async_copy(obuf.at[oslot],
                             out_h.at[pl.ds((base_row + CQ * ch) * C, CQ * C)],
                             sems_o[oslot])

        def out_wait(oslot):
            pltpu.make_async_copy(out_h.at[pl.ds(0, CQ * C)],
                                  obuf.at[oslot], sems_o[oslot]).wait()

        def compute(gslot, islot, oslot):
            def hh_body(hh, _):
                sq = hh // 8
                h = hh - 8 * sq
                bpos = sq * 768 + h * 24
                a0 = jnp.zeros((16,), jnp.float32)
                a1 = jnp.zeros((16,), jnp.float32)
                for cc in range(4):
                    wpos = sq * C + h * 32
                    wv0 = wbuf[islot, cc, pl.ds(wpos, 16)]
                    wv1 = wbuf[islot, cc, pl.ds(wpos + 16, 16)]
                    accp = jnp.zeros((2 * 16,), jnp.bfloat16)
                    for j in range(24):
                        pos = bpos + cc * PTS + j
                        w = wv0[j] if j < 16 else wv1[j - 16]
                        wsp = jnp.zeros((16,), jnp.float32) + w
                        wpk = plsc.pack(wsp, wsp,
                                        format=plsc.PackFormat.INTERLEAVED)
                        accp = accp + wpk * gbuf[gslot, pos, :]
                    lo, hi = plsc.unpack(accp,
                                         format=plsc.PackFormat.INTERLEAVED)
                    a0 = a0 + lo
                    a1 = a1 + hi
                obuf[oslot, pl.ds(sq * C + h * 32, 16)] = a0
                obuf[oslot, pl.ds(sq * C + h * 32 + 16, 16)] = a1
                return 0
            lax.fori_loop(0, CQ * 8, hh_body, 0)

        idx_issue(0, 0)
        idx_issue(1, 1)
        idx_wait(0)
        gather_issue(0, 0)

        def outer(g2, _):
            for s in range(4):
                g = g2 * 4 + s
                gslot = s % 2
                gslot1 = (s + 1) % 2
                islot1 = (s + 1) % 4
                islot2 = (s + 2) % 4
                gather_wait(gslot)

                @pl.when(g + 1 < NCH)
                def _():
                    idx_wait(islot1)
                    gather_issue(gslot1, islot1)

                @pl.when(g + 2 < NCH)
                def _():
                    idx_issue(g + 2, islot2)

                @pl.when(g >= 2)
                def _():
                    out_wait(gslot)

                compute(gslot, s, gslot)
                out_issue(g, gslot)
            return 0

        lax.fori_loop(0, NCH // 4, outer, 0)
        out_wait(0)
        out_wait(1)

    return k(table, idxr, *wrs)


def kernel(query, query_pos, value, reference_points, spatial_shapes,
           W_so, b_so, W_aw, b_aw, W_v, b_v, W_o, b_o):
    bs, nq, _ = query.shape
    ref12 = reference_points.reshape(bs, nq, 2 * ML)
    wsox = W_so[:, 0::2]
    wsoy = W_so[:, 1::2]
    bsox = b_so[0::2].reshape(1, PTS)
    bsoy = b_so[1::2].reshape(1, PTS)
    baw = b_aw.reshape(1, PTS)

    colperm = np.arange(C).reshape(NH, 2, 16).transpose(0, 2, 1).reshape(-1)
    vp = _vproj_call(value, W_v[:, colperm], b_v[colperm])
    table = vp.reshape(bs * NV * NH, D)

    outs = []
    qoff = 0
    for nqh in (nq // 8, nq // 4, 3 * nq // 8, nq // 4):
        idx_all, w0, w1, w2, w3 = _prep_call(
            query, query_pos, ref12, wsox, bsox, wsoy, bsoy, W_aw, baw,
            qoff, nqh)
        idxr = idx_all.reshape(-1)
        wrs = [w.reshape(-1) for w in (w0, w1, w2, w3)]
        attn = _sc_sample(table, idxr, wrs, bs * nqh).reshape(bs, nqh, C)
        outs.append(_final_call(attn, W_o, b_o, query, qoff, nqh))
        qoff += nqh
    return jnp.concatenate(outs, axis=1)


# speedup vs baseline: 1.0031x; 1.0031x over previous
import functools

import numpy as np
import jax
import jax.numpy as jnp
from jax import lax
from jax.experimental import pallas as pl
from jax.experimental.pallas import tpu as pltpu
from jax.experimental.pallas import tpu_sc as plsc

NH = 8
NP = 4
ML = 6
D = 32
C = 256
_SS = np.array([[64, 64], [32, 32], [16, 16], [64, 64], [32, 32], [16, 16]], np.int64)
_STARTS = np.concatenate([np.zeros(1, np.int64), np.cumsum(_SS[:, 0] * _SS[:, 1])[:-1]])
NV = int((_SS[:, 0] * _SS[:, 1]).sum())
PTS = NH * ML * NP
QB = 512
VB = 768

_pt = np.arange(PTS)
_h_of = _pt // (ML * NP)
_l_of = (_pt % (ML * NP)) // NP
_Wc = _SS[_l_of, 1].astype(np.float32)[None]
_Hc = _SS[_l_of, 0].astype(np.float32)[None]
_Wci = _SS[_l_of, 1].astype(np.int32)[None]
_Hci = _SS[_l_of, 0].astype(np.int32)[None]
_invW = (1.0 / _SS[_l_of, 1]).astype(np.float32)[None]
_invH = (1.0 / _SS[_l_of, 0]).astype(np.float32)[None]
_addc = (_STARTS[_l_of] * NH + _h_of).astype(np.int32)[None]
_Sx = np.zeros((2 * ML, PTS), np.float32)
_Sx[2 * _l_of, _pt] = 1.0
_Sy = np.zeros((2 * ML, PTS), np.float32)
_Sy[2 * _l_of + 1, _pt] = 1.0
_Bsum = (_h_of[:, None] == _h_of[None, :]).astype(np.float32)

_CORNERS = ((0, 0), (1, 0), (0, 1), (1, 1))
_FC = np.concatenate([_Wc, _Hc, _invW, _invH], axis=0)
_IC = np.concatenate([_Wci, _Hci, _addc, _Wci * NH], axis=0)
_Ppad = np.zeros((PTS, C), np.float32)
_Ppad[_pt, _h_of * 32 + (_pt % (ML * NP))] = 1.0


def _prep_math(q, ref12, wsox, bsox, wsoy, bsoy, waw, baw, b,
               bsum, sx, sy, fc, ic):
    sox = jnp.dot(q, wsox, preferred_element_type=jnp.float32) + bsox
    soy = jnp.dot(q, wsoy, preferred_element_type=jnp.float32) + bsoy
    logits = jnp.dot(q, waw, preferred_element_type=jnp.float32) + baw
    e = jnp.exp(logits)
    denom = jnp.dot(e, bsum, preferred_element_type=jnp.float32, precision=lax.Precision.HIGHEST)
    aw = e / denom
    refx = jnp.dot(ref12, sx, preferred_element_type=jnp.float32, precision=lax.Precision.HIGHEST)
    refy = jnp.dot(ref12, sy, preferred_element_type=jnp.float32, precision=lax.Precision.HIGHEST)
    Wc, Hc, invW, invH = fc[0:1], fc[1:2], fc[2:3], fc[3:4]
    Wci, Hci, addc0, Wci8 = ic[0:1], ic[1:2], ic[2:3], ic[3:4]
    x = (refx + sox * invW) * Wc - 0.5
    y = (refy + soy * invH) * Hc - 0.5
    x0f = jnp.floor(x)
    fx = x - x0f
    x0 = x0f.astype(jnp.int32)
    y0f = jnp.floor(y)
    fy = y - y0f
    y0 = y0f.astype(jnp.int32)
    vx0 = (x0f >= 0.0) & (x0f <= Wc - 1.0)
    vx1 = (x0f >= -1.0) & (x0f <= Wc - 2.0)
    vy0 = (y0f >= 0.0) & (y0f <= Hc - 1.0)
    vy1 = (y0f >= -1.0) & (y0f <= Hc - 2.0)
    wx0 = jnp.where(vx0, 1.0 - fx, 0.0)
    wx1 = jnp.where(vx1, fx, 0.0)
    awy0 = jnp.where(vy0, 1.0 - fy, 0.0) * aw
    awy1 = jnp.where(vy1, fy, 0.0) * aw
    addc = addc0 + b * np.int32(NV * NH)
    rt0 = jnp.clip(y0, 0, Hci - 1) * Wci8 + addc
    rt1 = jnp.clip(y0 + 1, 0, Hci - 1) * Wci8 + addc
    ct0 = jnp.left_shift(jnp.clip(x0, 0, Wci - 1), 3)
    ct1 = jnp.left_shift(jnp.clip(x0 + 1, 0, Wci - 1), 3)
    idx_c = [rt0 + ct0, rt0 + ct1, rt1 + ct0, rt1 + ct1]
    w_c = [wx0 * awy0, wx1 * awy0, wx0 * awy1, wx1 * awy1]
    return idx_c, w_c


def _prep_kernel(qr, qpr, refr, wsox, bsox, wsoy, bsoy, waw, baw,
                 bsum, sx, sy, fc, ic, ppad, idx_o, w_o0, w_o1, w_o2, w_o3):
    b = pl.program_id(0)
    q = qr[0] + qpr[0]
    idx_c, w_c = _prep_math(
        q, refr[0], wsox[...], bsox[...], wsoy[...], bsoy[...], waw[...],
        baw[...], b, bsum[...], sx[...], sy[...], fc[...], ic[...])
    w_os = (w_o0, w_o1, w_o2, w_o3)
    for c in range(4):
        idx_o[0, :, c, :] = idx_c[c]
        w_os[c][0] = jnp.dot(w_c[c], ppad[...],
                             preferred_element_type=jnp.float32)


def _prep_call(query, query_pos, ref12, wsox, bsox, wsoy, bsoy, waw, baw,
               qoff, nqh):
    bs = query.shape[0]
    ob = qoff // QB
    qspec = pl.BlockSpec((1, QB, C), lambda b, i: (b, i + ob, 0))
    wspec = pl.BlockSpec((C, PTS), lambda b, i: (0, 0))
    bspec = pl.BlockSpec((1, PTS), lambda b, i: (0, 0))
    cspec = pl.BlockSpec((4, PTS), lambda b, i: (0, 0))
    return pl.pallas_call(
        _prep_kernel,
        grid=(bs, nqh // QB),
        in_specs=[
            qspec, qspec,
            pl.BlockSpec((1, QB, 2 * ML), lambda b, i: (b, i + ob, 0)),
            wspec, bspec, wspec, bspec, wspec, bspec,
            pl.BlockSpec((PTS, PTS), lambda b, i: (0, 0)),
            pl.BlockSpec((2 * ML, PTS), lambda b, i: (0, 0)),
            pl.BlockSpec((2 * ML, PTS), lambda b, i: (0, 0)),
            cspec, cspec,
            pl.BlockSpec((PTS, C), lambda b, i: (0, 0)),
        ],
        out_specs=[
            pl.BlockSpec((1, QB, 4, PTS), lambda b, i: (b, i, 0, 0)),
        ] + [pl.BlockSpec((1, QB, C), lambda b, i: (b, i, 0))] * 4,
        out_shape=[
            jax.ShapeDtypeStruct((bs, nqh, 4, PTS), jnp.int32),
        ] + [jax.ShapeDtypeStruct((bs, nqh, C), jnp.float32)] * 4,
    )(query, query_pos, ref12, wsox, bsox, wsoy, bsoy, waw, baw,
      jnp.asarray(_Bsum), jnp.asarray(_Sx), jnp.asarray(_Sy),
      jnp.asarray(_FC), jnp.asarray(_IC), jnp.asarray(_Ppad))


def _vproj_kernel(vr, wv, bv, o):
    o[0] = (jnp.dot(vr[0], wv[...], preferred_element_type=jnp.float32)
            + bv[...]).astype(jnp.bfloat16)


def _vproj_call(value, W_v, b_v):
    bs, nv, _ = value.shape
    return pl.pallas_call(
        _vproj_kernel,
        grid=(bs, nv // VB),
        in_specs=[
            pl.BlockSpec((1, VB, C), lambda b, i: (b, i, 0)),
            pl.BlockSpec((C, C), lambda b, i: (0, 0)),
            pl.BlockSpec((1, C), lambda b, i: (0, 0)),
        ],
        out_specs=pl.BlockSpec((1, VB, C), lambda b, i: (b, i, 0)),
        out_shape=jax.ShapeDtypeStruct((bs, nv, C), jnp.bfloat16),
    )(value, W_v, b_v.reshape(1, C))


def _final_kernel(ar, wo, bo, qr, o):
    o[0] = (jnp.dot(ar[0], wo[...], preferred_element_type=jnp.float32)
            + bo[...] + qr[0])


def _final_call(attn, W_o, b_o, query, qoff, nqh):
    bs = query.shape[0]
    ob = qoff // QB
    spec = pl.BlockSpec((1, QB, C), lambda b, i: (b, i, 0))
    return pl.pallas_call(
        _final_kernel,
        grid=(bs, nqh // QB),
        in_specs=[
            spec,
            pl.BlockSpec((C, C), lambda b, i: (0, 0)),
            pl.BlockSpec((1, C), lambda b, i: (0, 0)),
            pl.BlockSpec((1, QB, C), lambda b, i: (b, i + ob, 0)),
        ],
        out_specs=spec,
        out_shape=jax.ShapeDtypeStruct((bs, nqh, C), jnp.float32),
    )(attn, W_o, b_o.reshape(1, C), query)


def _sc_sample(table, idxr, wrs, bs_nq):
    CQ = 4
    RPT = bs_nq // 32
    NCH = RPT // CQ
    ROWS = CQ * 4 * PTS
    mesh = plsc.VectorSubcoreMesh(core_axis_name="c", subcore_axis_name="s")

    @functools.partial(
        pl.kernel,
        out_type=jax.ShapeDtypeStruct((bs_nq * C,), jnp.float32),
        mesh=mesh,
        compiler_params=pltpu.CompilerParams(use_tc_tiling_on_sc=False,
                                             needs_layout_passes=False),
        scratch_types=[
            pltpu.VMEM((2, ROWS, D), jnp.bfloat16),
            pltpu.VMEM((4, ROWS), jnp.int32),
            pltpu.VMEM((4, 4, CQ * C), jnp.float32),
            pltpu.VMEM((2, CQ * C), jnp.float32),
            pltpu.SemaphoreType.DMA,
            pltpu.SemaphoreType.DMA,
            pltpu.SemaphoreType.DMA,
            pltpu.SemaphoreType.DMA,
            pltpu.SemaphoreType.DMA,
            pltpu.SemaphoreType.DMA,
            pltpu.SemaphoreType.DMA,
            pltpu.SemaphoreType.DMA,
        ],
    )
    def k(table_h, idx_h, w0_h, w1_h, w2_h, w3_h, out_h, gbuf, ibuf, wbuf,
          obuf, si0, si1, si2, si3, sg0, sg1, so0, so1):
        w_hs = (w0_h, w1_h, w2_h, w3_h)
        sems_i = (si0, si1, si2, si3)
        sems_g = (sg0, sg1)
        sems_o = (so0, so1)
        wid = lax.axis_index("s") * 2 + lax.axis_index("c")
        base_row = wid * RPT

        def idx_issue(ch, islot):
            pltpu.async_copy(idx_h.at[pl.ds((base_row + CQ * ch) * 768, ROWS)],
                             ibuf.at[islot], sems_i[islot])
            for cc in range(4):
                pltpu.async_copy(
                    w_hs[cc].at[pl.ds((base_row + CQ * ch) * C, CQ * C)],
                    wbuf.at[islot, cc], sems_i[islot])

        def idx_wait(islot):
            pltpu.make_async_copy(idx_h.at[pl.ds(0, ROWS)],
                                  ibuf.at[islot], sems_i[islot]).wait()
            for cc in range(4):
                pltpu.make_async_copy(w_hs[cc].at[pl.ds(0, 2 * C)],
                                      wbuf.at[islot, cc],
                                      sems_i[islot]).wait()

        def gather_issue(gslot, islot):
            def gi(kk, _):
                pltpu.async_copy(table_h.at[ibuf.at[islot, pl.ds(kk * 128, 128)]],
                                 gbuf.at[gslot, pl.ds(kk * 128, 128), :],
                                 sems_g[gslot])
                return 0
            lax.fori_loop(0, ROWS // 128, gi, 0)

        def gather_wait(gslot):
            pltpu.make_async_copy(table_h.at[pl.ds(0, ROWS), :],
                                  gbuf.at[gslot], sems_g[gslot]).wait()

        def out_issue(ch, oslot):
            pltpu.async_copy(obuf.at[oslot],
                             out_h.at[pl.ds((base_row + CQ * ch) * C, CQ * C)],
                             sems_o[oslot])

        def out_wait(oslot):
            pltpu.make_async_copy(out_h.at[pl.ds(0, CQ * C)],
                                  obuf.at[oslot], sems_o[oslot]).wait()

        def compute(gslot, islot, oslot):
            def hh_body(hh, _):
                sq = hh // 8
                h = hh - 8 * sq
                bpos = sq * 768 + h * 24
                a0 = jnp.zeros((16,), jnp.float32)
                a1 = jnp.zeros((16,), jnp.float32)
                for cc in range(4):
                    wpos = sq * C + h * 32
                    wv0 = wbuf[islot, cc, pl.ds(wpos, 16)]
                    wv1 = wbuf[islot, cc, pl.ds(wpos + 16, 16)]
                    accp = jnp.zeros((2 * 16,), jnp.bfloat16)
                    for j in range(24):
                        pos = bpos + cc * PTS + j
                        w = wv0[j] if j < 16 else wv1[j - 16]
                        wsp = jnp.zeros((16,), jnp.float32) + w
                        wpk = plsc.pack(wsp, wsp,
                                        format=plsc.PackFormat.INTERLEAVED)
                        accp = accp + wpk * gbuf[gslot, pos, :]
                    lo, hi = plsc.unpack(accp,
                                         format=plsc.PackFormat.INTERLEAVED)
                    a0 = a0 + lo
                    a1 = a1 + hi
                obuf[oslot, pl.ds(sq * C + h * 32, 16)] = a0
                obuf[oslot, pl.ds(sq * C + h * 32 + 16, 16)] = a1
                return 0
            lax.fori_loop(0, CQ * 8, hh_body, 0)

        idx_issue(0, 0)
        idx_issue(1, 1)
        idx_wait(0)
        gather_issue(0, 0)

        def outer(g2, _):
            for s in range(4):
                g = g2 * 4 + s
                gslot = s % 2
                gslot1 = (s + 1) % 2
                islot1 = (s + 1) % 4
                islot2 = (s + 2) % 4
                gather_wait(gslot)

                @pl.when(g + 1 < NCH)
                def _():
                    idx_wait(islot1)
                    gather_issue(gslot1, islot1)

                @pl.when(g + 2 < NCH)
                def _():
                    idx_issue(g + 2, islot2)

                @pl.when(g >= 2)
                def _():
                    out_wait(gslot)

                compute(gslot, s, gslot)
                out_issue(g, gslot)
            return 0

        lax.fori_loop(0, NCH // 4, outer, 0)
        out_wait(0)
        out_wait(1)

    return k(table, idxr, *wrs)


def kernel(query, query_pos, value, reference_points, spatial_shapes,
           W_so, b_so, W_aw, b_aw, W_v, b_v, W_o, b_o):
    bs, nq, _ = query.shape
    ref12 = reference_points.reshape(bs, nq, 2 * ML)
    wsox = W_so[:, 0::2]
    wsoy = W_so[:, 1::2]
    bsox = b_so[0::2].reshape(1, PTS)
    bsoy = b_so[1::2].reshape(1, PTS)
    baw = b_aw.reshape(1, PTS)

    colperm = np.arange(C).reshape(NH, 2, 16).transpose(0, 2, 1).reshape(-1)
    vp = _vproj_call(value, W_v[:, colperm], b_v[colperm])
    table = vp.reshape(bs * NV * NH, D)

    outs = []
    qoff = 0
    for nqh in (nq // 8, 3 * nq // 8, 3 * nq // 8, nq // 8):
        idx_all, w0, w1, w2, w3 = _prep_call(
            query, query_pos, ref12, wsox, bsox, wsoy, bsoy, W_aw, baw,
            qoff, nqh)
        idxr = idx_all.reshape(-1)
        wrs = [w.reshape(-1) for w in (w0, w1, w2, w3)]
        attn = _sc_sample(table, idxr, wrs, bs * nqh).reshape(bs, nqh, C)
        outs.append(_final_call(attn, W_o, b_o, query, qoff, nqh))
        qoff += nqh
    return jnp.concatenate(outs, axis=1)
